# baseline (speedup 1.0000x reference)
import functools

import jax
import jax.numpy as jnp
from jax import lax
from jax.experimental import pallas as pl
from jax.experimental.pallas import tpu as pltpu
from jax.experimental.pallas import tpu_sc as plsc

BATCH = 16384
VOCAB = 1000000

NUM_CORES = 2
NUM_SUBCORES = 16
NUM_WORKERS = NUM_CORES * NUM_SUBCORES
B_PER_W = BATCH // NUM_WORKERS
CHUNK = 128
N_CHUNKS = B_PER_W // CHUNK
LANES = 16


def _sc_body(idx_hbm, pos_hbm, neg_hbm, out_hbm, idx_v, pos_v, neg_v, out_v, sem):
    wid = lax.axis_index("s") * NUM_CORES + lax.axis_index("c")
    base = wid * B_PER_W

    pltpu.sync_copy(idx_hbm.at[pl.ds(base, B_PER_W)], idx_v)

    copies = []
    for c in range(N_CHUNKS):
        sl = pl.ds(c * CHUNK, CHUNK)
        copies.append(pltpu.async_copy(pos_hbm.at[idx_v.at[sl]], pos_v.at[sl], sem))
        copies.append(pltpu.async_copy(neg_hbm.at[idx_v.at[sl]], neg_v.at[sl], sem))
    for cp in copies:
        cp.wait()

    iota3 = lax.iota(jnp.int32, LANES) * 3
    for j in range(B_PER_W // LANES):
        p = pos_v[pl.ds(j * LANES, LANES)]
        n = neg_v[pl.ds(j * LANES, LANES)]
        prob = p / (p + n)
        fidx = iota3 + (j * LANES * 3)
        plsc.store_scatter(out_v, [fidx], p)
        plsc.store_scatter(out_v, [fidx + 1], n)
        plsc.store_scatter(out_v, [fidx + 2], prob)

    pltpu.sync_copy(out_v, out_hbm.at[pl.ds(base * 3, B_PER_W * 3)])


@functools.partial(
    pl.kernel,
    out_type=jax.ShapeDtypeStruct((BATCH * 3,), jnp.float32),
    mesh=plsc.VectorSubcoreMesh(core_axis_name="c", subcore_axis_name="s"),
    compiler_params=pltpu.CompilerParams(
        needs_layout_passes=False, skip_device_barrier=True
    ),
    scratch_types=[
        pltpu.VMEM((B_PER_W,), jnp.int32),
        pltpu.VMEM((B_PER_W,), jnp.float32),
        pltpu.VMEM((B_PER_W,), jnp.float32),
        pltpu.VMEM((B_PER_W * 3,), jnp.float32),
        pltpu.SemaphoreType.DMA,
    ],
)
def _encode(idx_hbm, pos_hbm, neg_hbm, out_hbm, idx_v, pos_v, neg_v, out_v, sem):
    _sc_body(idx_hbm, pos_hbm, neg_hbm, out_hbm, idx_v, pos_v, neg_v, out_v, sem)


def kernel(inputs, positive_frequency_lookup, negative_frequency_lookup):
    idx = inputs.reshape(BATCH)
    pos_t = positive_frequency_lookup.reshape(VOCAB)
    neg_t = negative_frequency_lookup.reshape(VOCAB)
    out_flat = _encode(idx, pos_t, neg_t)
    return out_flat.reshape(BATCH, 3)


# speedup vs baseline: 1.1378x; 1.1378x over previous
import functools

import jax
import jax.numpy as jnp
from jax import lax
from jax.experimental import pallas as pl
from jax.experimental.pallas import tpu as pltpu
from jax.experimental.pallas import tpu_sc as plsc

BATCH = 16384
VOCAB = 1000000

NUM_CORES = 2
NUM_SUBCORES = 16
NUM_WORKERS = NUM_CORES * NUM_SUBCORES
B_PER_W = BATCH // NUM_WORKERS
CHUNK = 128
N_CHUNKS = B_PER_W // CHUNK
LANES = 16


def _sc_body(
    idx_hbm, pos_hbm, neg_hbm, outp_hbm, outn_hbm, outq_hbm,
    idx_v, pos_v, neg_v, prob_v, sem,
):
    wid = lax.axis_index("s") * NUM_CORES + lax.axis_index("c")
    base = wid * B_PER_W

    pltpu.sync_copy(idx_hbm.at[pl.ds(base, B_PER_W)], idx_v)

    copies = []
    for c in range(N_CHUNKS):
        sl = pl.ds(c * CHUNK, CHUNK)
        copies.append(pltpu.async_copy(pos_hbm.at[idx_v.at[sl]], pos_v.at[sl], sem))
        copies.append(pltpu.async_copy(neg_hbm.at[idx_v.at[sl]], neg_v.at[sl], sem))
    for cp in copies:
        cp.wait()

    for j in range(B_PER_W // LANES):
        sl = pl.ds(j * LANES, LANES)
        p = pos_v[sl]
        n = neg_v[sl]
        prob_v[sl] = p / (p + n)

    pltpu.sync_copy(pos_v, outp_hbm.at[pl.ds(base, B_PER_W)])
    pltpu.sync_copy(neg_v, outn_hbm.at[pl.ds(base, B_PER_W)])
    pltpu.sync_copy(prob_v, outq_hbm.at[pl.ds(base, B_PER_W)])


@functools.partial(
    pl.kernel,
    out_type=(
        jax.ShapeDtypeStruct((BATCH,), jnp.float32),
        jax.ShapeDtypeStruct((BATCH,), jnp.float32),
        jax.ShapeDtypeStruct((BATCH,), jnp.float32),
    ),
    mesh=plsc.VectorSubcoreMesh(core_axis_name="c", subcore_axis_name="s"),
    compiler_params=pltpu.CompilerParams(
        needs_layout_passes=False,
        skip_device_barrier=True,
        use_tc_tiling_on_sc=False,
    ),
    scratch_types=[
        pltpu.VMEM((B_PER_W,), jnp.int32),
        pltpu.VMEM((B_PER_W,), jnp.float32),
        pltpu.VMEM((B_PER_W,), jnp.float32),
        pltpu.VMEM((B_PER_W,), jnp.float32),
        pltpu.SemaphoreType.DMA,
    ],
)
def _encode(
    idx_hbm, pos_hbm, neg_hbm, outp_hbm, outn_hbm, outq_hbm,
    idx_v, pos_v, neg_v, prob_v, sem,
):
    _sc_body(
        idx_hbm, pos_hbm, neg_hbm, outp_hbm, outn_hbm, outq_hbm,
        idx_v, pos_v, neg_v, prob_v, sem,
    )


def kernel(inputs, positive_frequency_lookup, negative_frequency_lookup):
    idx = inputs.reshape(BATCH)
    pos_t = positive_frequency_lookup.reshape(VOCAB)
    neg_t = negative_frequency_lookup.reshape(VOCAB)
    p, n, q = _encode(idx, pos_t, neg_t)
    return jnp.concatenate([p[:, None], n[:, None], q[:, None]], axis=1)


# speedup vs baseline: 3.3533x; 2.9473x over previous
import functools

import jax
import jax.numpy as jnp
from jax import lax
from jax.experimental import pallas as pl
from jax.experimental.pallas import tpu as pltpu
from jax.experimental.pallas import tpu_sc as plsc

BATCH = 16384
VOCAB = 1000000
VOCAB_PAD = 1000448

NUM_CORES = 2
NUM_SUBCORES = 16
NUM_WORKERS = NUM_CORES * NUM_SUBCORES
B_PER_W = BATCH // NUM_WORKERS
CHUNK = 128
N_CHUNKS = B_PER_W // CHUNK
LANES = 16


def _sc_body(
    idx_hbm, pos_hbm, neg_hbm, outp_hbm, outn_hbm, outq_hbm,
    idx_v, pos_v, neg_v, prob_v, sem,
):
    wid = lax.axis_index("s") * NUM_CORES + lax.axis_index("c")
    base = wid * B_PER_W

    pltpu.sync_copy(idx_hbm.at[pl.ds(base, B_PER_W)], idx_v)

    copies = []
    for c in range(N_CHUNKS):
        sl = pl.ds(c * CHUNK, CHUNK)
        copies.append(pltpu.async_copy(pos_hbm.at[idx_v.at[sl]], pos_v.at[sl], sem))
        copies.append(pltpu.async_copy(neg_hbm.at[idx_v.at[sl]], neg_v.at[sl], sem))
    for cp in copies:
        cp.wait()

    for j in range(B_PER_W // LANES):
        sl = pl.ds(j * LANES, LANES)
        p = pos_v[sl]
        n = neg_v[sl]
        prob_v[sl] = p / (p + n)

    pltpu.sync_copy(pos_v, outp_hbm.at[pl.ds(base, B_PER_W)])
    pltpu.sync_copy(neg_v, outn_hbm.at[pl.ds(base, B_PER_W)])
    pltpu.sync_copy(prob_v, outq_hbm.at[pl.ds(base, B_PER_W)])


@functools.partial(
    pl.kernel,
    out_type=(
        jax.ShapeDtypeStruct((BATCH,), jnp.float32),
        jax.ShapeDtypeStruct((BATCH,), jnp.float32),
        jax.ShapeDtypeStruct((BATCH,), jnp.float32),
    ),
    mesh=plsc.VectorSubcoreMesh(core_axis_name="c", subcore_axis_name="s"),
    compiler_params=pltpu.CompilerParams(
        needs_layout_passes=False,
        skip_device_barrier=True,
        use_tc_tiling_on_sc=False,
    ),
    scratch_types=[
        pltpu.VMEM((B_PER_W,), jnp.int32),
        pltpu.VMEM((B_PER_W,), jnp.float32),
        pltpu.VMEM((B_PER_W,), jnp.float32),
        pltpu.VMEM((B_PER_W,), jnp.float32),
        pltpu.SemaphoreType.DMA,
    ],
)
def _encode(
    idx_hbm, pos_hbm, neg_hbm, outp_hbm, outn_hbm, outq_hbm,
    idx_v, pos_v, neg_v, prob_v, sem,
):
    _sc_body(
        idx_hbm, pos_hbm, neg_hbm, outp_hbm, outn_hbm, outq_hbm,
        idx_v, pos_v, neg_v, prob_v, sem,
    )


def kernel(inputs, positive_frequency_lookup, negative_frequency_lookup):
    idx = inputs.reshape(BATCH)
    pad = ((0, VOCAB_PAD - VOCAB), (0, 0))
    pos_t = jnp.pad(positive_frequency_lookup, pad).reshape(VOCAB_PAD)
    neg_t = jnp.pad(negative_frequency_lookup, pad).reshape(VOCAB_PAD)
    p, n, q = _encode(idx, pos_t, neg_t)
    return jnp.concatenate([p[:, None], n[:, None], q[:, None]], axis=1)
